# Initial kernel scaffold; baseline (speedup 1.0000x reference)
import functools
import math

import jax
import jax.numpy as jnp
from jax import lax
from jax.experimental import pallas as pl
from jax.experimental.pallas import tpu as pltpu
from jax.experimental.pallas import tpu_sc as plsc

N = 10000
E = 320000
DIM = 128
HEADS = 8
DK = DIM // HEADS
EDGE_DIM = 16

NC = 2
NS = 16
NW = NC * NS
EW = E // NW
BLK = 80
NBLK = EW // BLK
SUB = BLK // 16
ACCW = 144
ROWS_PER_TILE = N // NS

_HIGH = jax.lax.Precision.HIGHEST


def _dotT(a, b):
    return lax.dot_general(a, b, (((1,), (1,)), ((), ())),
                           precision=_HIGH, preferred_element_type=jnp.float32)


def _proj_body(x_ref, wq_ref, bq_ref, wk_ref, bk_ref, wv_ref, bv_ref,
               q_ref, k_ref, v_ref):
    xb = x_ref[...]
    scale = 1.0 / math.sqrt(DK)
    q_ref[...] = (_dotT(xb, wq_ref[...]) + bq_ref[...][None, :]) * scale
    k_ref[...] = _dotT(xb, wk_ref[...]) + bk_ref[...][None, :]
    v_ref[...] = _dotT(xb, wv_ref[...]) + bv_ref[...][None, :]


def _proj(x, WQ, bQ, WK, bK, WV, bV):
    nb = 10
    blk = N // nb
    w_spec = pl.BlockSpec((DIM, DIM), lambda i: (0, 0))
    b_spec = pl.BlockSpec((DIM,), lambda i: (0,))
    row_spec = pl.BlockSpec((blk, DIM), lambda i: (i, 0))
    out = jax.ShapeDtypeStruct((N, DIM), jnp.float32)
    return pl.pallas_call(
        _proj_body,
        grid=(nb,),
        in_specs=[row_spec, w_spec, b_spec, w_spec, b_spec, w_spec, b_spec],
        out_specs=[row_spec, row_spec, row_spec],
        out_shape=[out, out, out],
    )(x, WQ, bQ, WK, bK, WV, bV)


def _bias_body(ea_ref, w1_ref, b1_ref, w2_ref, b2_ref, o_ref):
    h = _dotT(ea_ref[...], w1_ref[...]) + b1_ref[...][None, :]
    h = h * (1.0 / (1.0 + jnp.exp(-h)))
    o_ref[...] = _dotT(h, w2_ref[...]) + b2_ref[...][None, :]


def _edge_bias(edge_attr, W1, b1, W2, b2):
    nb = 80
    blk = E // nb
    return pl.pallas_call(
        _bias_body,
        grid=(nb,),
        in_specs=[
            pl.BlockSpec((blk, EDGE_DIM), lambda i: (i, 0)),
            pl.BlockSpec((EDGE_DIM, EDGE_DIM), lambda i: (0, 0)),
            pl.BlockSpec((EDGE_DIM,), lambda i: (0,)),
            pl.BlockSpec((HEADS, EDGE_DIM), lambda i: (0, 0)),
            pl.BlockSpec((HEADS,), lambda i: (0,)),
        ],
        out_specs=pl.BlockSpec((blk, HEADS), lambda i: (i, 0)),
        out_shape=jax.ShapeDtypeStruct((E, HEADS), jnp.float32),
    )(edge_attr, W1, b1, W2, b2)


def _sc_body(q_hbm, k_hbm, v_hbm, bias_hbm, i_hbm, j_hbm, out_hbm,
             i_v, j_v, q_v, k_v, v_v, bias_v, stage, zbuf, acc,
             sem0, sem1, sem2):
    cid = lax.axis_index("c")
    sid = lax.axis_index("s")
    wid = cid * NS + sid

    z16 = jnp.zeros((16,), jnp.float32)

    @pl.loop(0, BLK)
    def _(r):
        for c in range(ACCW // 16):
            stage[r, pl.ds(c * 16, 16)] = z16

    @pl.loop(0, 25)
    def _(r):
        for c in range(ACCW // 16):
            zbuf[r, pl.ds(c * 16, 16)] = z16

    @pl.loop(0, ROWS_PER_TILE // 25)
    def _(b):
        pltpu.sync_copy(zbuf, acc.at[pl.ds(sid * ROWS_PER_TILE + b * 25, 25)])

    plsc.subcore_barrier()

    wbase = wid * EW

    @pl.loop(0, NBLK)
    def _(blk):
        base = wbase + blk * BLK
        pltpu.sync_copy(i_hbm.at[pl.ds(base, BLK)], i_v)
        pltpu.sync_copy(j_hbm.at[pl.ds(base, BLK)], j_v)
        pltpu.sync_copy(bias_hbm.at[pl.ds(base, BLK)], bias_v)
        cq = pltpu.async_copy(q_hbm.at[j_v], q_v, sem0)
        ck = pltpu.async_copy(k_hbm.at[i_v], k_v, sem1)
        cv = pltpu.async_copy(v_hbm.at[i_v], v_v, sem2)
        cq.wait()
        ck.wait()
        cv.wait()

        @pl.loop(0, SUB)
        def _(sb):
            eidx = sb * 16 + lax.iota(jnp.int32, 16)
            for h in range(HEADS):
                acc16 = z16
                for dd in range(DK):
                    col = jnp.full((16,), h * DK + dd, jnp.int32)
                    qv = plsc.load_gather(q_v, [eidx, col])
                    kv = plsc.load_gather(k_v, [eidx, col])
                    acc16 = acc16 + qv * kv
                bv = plsc.load_gather(bias_v, [eidx, jnp.full((16,), h, jnp.int32)])
                ex = jnp.exp(acc16 + bv)
                plsc.store_scatter(stage, [eidx, jnp.full((16,), DIM + h, jnp.int32)], ex)
                for dd in range(DK):
                    col = jnp.full((16,), h * DK + dd, jnp.int32)
                    vv = plsc.load_gather(v_v, [eidx, col])
                    plsc.store_scatter(stage, [eidx, col], ex * vv)

        pltpu.sync_copy(stage, acc.at[j_v], add=True)

    plsc.subcore_barrier()

    pltpu.sync_copy(acc.at[pl.ds(sid * ROWS_PER_TILE, ROWS_PER_TILE)],
                    out_hbm.at[cid, pl.ds(sid * ROWS_PER_TILE, ROWS_PER_TILE)])


_sc_edge_pass = functools.partial(
    pl.kernel,
    out_type=jax.ShapeDtypeStruct((NC, N, ACCW), jnp.float32),
    mesh=plsc.VectorSubcoreMesh(core_axis_name="c", subcore_axis_name="s"),
    scratch_types=[
        pltpu.VMEM((BLK,), jnp.int32),
        pltpu.VMEM((BLK,), jnp.int32),
        pltpu.VMEM((BLK, DIM), jnp.float32),
        pltpu.VMEM((BLK, DIM), jnp.float32),
        pltpu.VMEM((BLK, DIM), jnp.float32),
        pltpu.VMEM((BLK, HEADS), jnp.float32),
        pltpu.VMEM((BLK, ACCW), jnp.float32),
        pltpu.VMEM((25, ACCW), jnp.float32),
        pltpu.VMEM_SHARED((N, ACCW), jnp.float32),
        pltpu.SemaphoreType.DMA,
        pltpu.SemaphoreType.DMA,
        pltpu.SemaphoreType.DMA,
    ],
)(_sc_body)


def _final_body(n0_ref, n1_ref, d0_ref, d1_ref, wo_ref, bo_ref, o_ref):
    node = n0_ref[...] + n1_ref[...]
    den = d0_ref[...] + d1_ref[...]
    rr = (lax.broadcasted_iota(jnp.int32, (HEADS, DIM), 1) // DK
          == lax.broadcasted_iota(jnp.int32, (HEADS, DIM), 0)
          ).astype(jnp.float32)
    den_exp = lax.dot_general(den, rr, (((1,), (0,)), ((), ())),
                              precision=_HIGH,
                              preferred_element_type=jnp.float32)
    norm = node / (den_exp + 1e-16)
    o_ref[...] = _dotT(norm, wo_ref[...]) + bo_ref[...][None, :]


def _final(n0, n1, d0, d1, WO, bO):
    nb = 10
    blk = N // nb
    return pl.pallas_call(
        _final_body,
        grid=(nb,),
        in_specs=[
            pl.BlockSpec((blk, DIM), lambda i: (i, 0)),
            pl.BlockSpec((blk, DIM), lambda i: (i, 0)),
            pl.BlockSpec((blk, HEADS), lambda i: (i, 0)),
            pl.BlockSpec((blk, HEADS), lambda i: (i, 0)),
            pl.BlockSpec((DIM, DIM), lambda i: (0, 0)),
            pl.BlockSpec((DIM,), lambda i: (0,)),
        ],
        out_specs=pl.BlockSpec((blk, DIM), lambda i: (i, 0)),
        out_shape=jax.ShapeDtypeStruct((N, DIM), jnp.float32),
    )(n0, n1, d0, d1, WO, bO)


def kernel(x, edge_index, edge_attr, WQ, bQ, WK, bK, WV, bV, WO, bO,
           W1, b1, W2, b2):
    qs, ks, vs = _proj(x, WQ, bQ, WK, bK, WV, bV)
    bias = _edge_bias(edge_attr, W1, b1, W2, b2)
    i_idx = edge_index[0]
    j_idx = edge_index[1]
    partials = _sc_edge_pass(qs, ks, vs, bias, i_idx, j_idx)
    n0 = partials[0, :, :DIM]
    n1 = partials[1, :, :DIM]
    d0 = partials[0, :, DIM:DIM + HEADS]
    d1 = partials[1, :, DIM:DIM + HEADS]
    return _final(n0, n1, d0, d1, WO, bO)


# speedup vs baseline: 1.5310x; 1.5310x over previous
import dataclasses
import functools
import math

import jax
import jax.numpy as jnp
from jax import lax
from jax.experimental import pallas as pl
from jax.experimental.pallas import tpu as pltpu
from jax.experimental.pallas import tpu_sc as plsc

N = 10000
E = 320000
DIM = 128
HEADS = 8
DK = DIM // HEADS
EDGE_DIM = 16

NC = 2
NS = 16
NW = NC * NS
EW = E // NW
BLK = 80
NBLK = EW // BLK
SUB = BLK // 16
N_PAD = 10240
ROWS_PER_TILE = N_PAD // NS
NDEN = N_PAD // 16
DEN_PER_TILE = NDEN // NS

_HIGH = jax.lax.Precision.HIGHEST


def _dotT(a, b):
    return lax.dot_general(a, b, (((1,), (1,)), ((), ())),
                           precision=_HIGH, preferred_element_type=jnp.float32)


def _proj_body(x_ref, wq_ref, bq_ref, wk_ref, bk_ref, wv_ref, bv_ref,
               q_ref, k_ref, v_ref):
    xb = x_ref[...]
    scale = 1.0 / math.sqrt(DK)
    q_ref[...] = (_dotT(xb, wq_ref[...]) + bq_ref[...][None, :]) * scale
    k_ref[...] = _dotT(xb, wk_ref[...]) + bk_ref[...][None, :]
    v_ref[...] = _dotT(xb, wv_ref[...]) + bv_ref[...][None, :]


def _proj(x, WQ, bQ, WK, bK, WV, bV):
    nb = 10
    blk = N // nb
    w_spec = pl.BlockSpec((DIM, DIM), lambda i: (0, 0))
    b_spec = pl.BlockSpec((DIM,), lambda i: (0,))
    row_spec = pl.BlockSpec((blk, DIM), lambda i: (i, 0))
    out = jax.ShapeDtypeStruct((N, DIM), jnp.float32)
    return pl.pallas_call(
        _proj_body,
        grid=(nb,),
        in_specs=[row_spec, w_spec, b_spec, w_spec, b_spec, w_spec, b_spec],
        out_specs=[row_spec, row_spec, row_spec],
        out_shape=[out, out, out],
    )(x, WQ, bQ, WK, bK, WV, bV)


def _bias_body(ea_ref, w1_ref, b1_ref, w2_ref, b2_ref, o_ref):
    h = _dotT(ea_ref[...], w1_ref[...]) + b1_ref[...][None, :]
    h = h * (1.0 / (1.0 + jnp.exp(-h)))
    o_ref[...] = _dotT(h, w2_ref[...]) + b2_ref[...][None, :]


def _edge_bias(edge_attr, W1, b1, W2, b2):
    nb = 80
    blk = E // nb
    return pl.pallas_call(
        _bias_body,
        grid=(nb,),
        in_specs=[
            pl.BlockSpec((blk, EDGE_DIM), lambda i: (i, 0)),
            pl.BlockSpec((EDGE_DIM, EDGE_DIM), lambda i: (0, 0)),
            pl.BlockSpec((EDGE_DIM,), lambda i: (0,)),
            pl.BlockSpec((HEADS, EDGE_DIM), lambda i: (0, 0)),
            pl.BlockSpec((HEADS,), lambda i: (0,)),
        ],
        out_specs=pl.BlockSpec((blk, HEADS), lambda i: (i, 0)),
        out_shape=jax.ShapeDtypeStruct((E, HEADS), jnp.float32),
    )(edge_attr, W1, b1, W2, b2)


def _sc_body(q_hbm, k_hbm, v_hbm, bias_hbm, i_hbm, j_hbm,
             out_hbm, outden_hbm,
             i_v, j_v, jdiv_v, q_v, k_v, bias_v, stage_den,
             acc, acc_den, sem0, sem1):
    cid = lax.axis_index("c")
    sid = lax.axis_index("s")
    wid = cid * NS + sid

    z16 = jnp.zeros((16,), jnp.float32)

    @pl.loop(0, BLK)
    def _(r):
        for c in range(DIM // 16):
            stage_den[r, pl.ds(c * 16, 16)] = z16

    @pl.loop(0, ROWS_PER_TILE // BLK)
    def _(b):
        pltpu.sync_copy(stage_den,
                        acc.at[pl.ds(sid * ROWS_PER_TILE + b * BLK, BLK)])

    pltpu.sync_copy(stage_den.at[pl.ds(0, DEN_PER_TILE)],
                    acc_den.at[pl.ds(sid * DEN_PER_TILE, DEN_PER_TILE)])

    plsc.subcore_barrier()

    wbase = wid * EW

    @pl.loop(0, NBLK)
    def _(blk):
        base = wbase + blk * BLK
        pltpu.sync_copy(i_hbm.at[pl.ds(base, BLK)], i_v)
        pltpu.sync_copy(j_hbm.at[pl.ds(base, BLK)], j_v)
        pltpu.sync_copy(bias_hbm.at[pl.ds(base, BLK)], bias_v)
        cq = pltpu.async_copy(q_hbm.at[j_v], q_v, sem0)
        ck = pltpu.async_copy(k_hbm.at[i_v], k_v, sem1)

        @pl.loop(0, SUB)
        def _(sb):
            sl = pl.ds(sb * 16, 16)
            jdiv_v[sl] = lax.shift_right_logical(j_v[sl], 4)

        cq.wait()
        ck.wait()

        @pl.loop(0, SUB)
        def _(sb):
            eidx = sb * 16 + lax.iota(jnp.int32, 16)
            j16 = plsc.load_gather(j_v, [eidx])
            colbase = lax.shift_left(jnp.bitwise_and(j16, 15), 3)
            for h in range(HEADS):
                acc16 = z16
                for dd in range(DK):
                    col = jnp.full((16,), h * DK + dd, jnp.int32)
                    qv = plsc.load_gather(q_v, [eidx, col])
                    kv = plsc.load_gather(k_v, [eidx, col])
                    acc16 = acc16 + qv * kv
                hcol = jnp.full((16,), h, jnp.int32)
                bv = plsc.load_gather(bias_v, [eidx, hcol])
                ex = jnp.exp(acc16 + bv)
                plsc.store_scatter(bias_v, [eidx, hcol], ex)
                plsc.store_scatter(stage_den, [eidx, colbase + h], ex)

        cv = pltpu.async_copy(v_hbm.at[i_v], q_v, sem0)
        pltpu.sync_copy(stage_den, acc_den.at[jdiv_v], add=True)
        cv.wait()

        @pl.loop(0, SUB)
        def _(sb):
            eidx = sb * 16 + lax.iota(jnp.int32, 16)
            for h in range(HEADS):
                ex = plsc.load_gather(bias_v, [eidx, jnp.full((16,), h, jnp.int32)])
                for dd in range(DK):
                    col = jnp.full((16,), h * DK + dd, jnp.int32)
                    vv = plsc.load_gather(q_v, [eidx, col])
                    plsc.store_scatter(k_v, [eidx, col], ex * vv)

        pltpu.sync_copy(k_v, acc.at[j_v], add=True)

        @pl.loop(0, SUB)
        def _(sb):
            eidx = sb * 16 + lax.iota(jnp.int32, 16)
            j16 = plsc.load_gather(j_v, [eidx])
            colbase = lax.shift_left(jnp.bitwise_and(j16, 15), 3)
            for h in range(HEADS):
                plsc.store_scatter(stage_den, [eidx, colbase + h], z16)

    plsc.subcore_barrier()

    pltpu.sync_copy(acc.at[pl.ds(sid * ROWS_PER_TILE, ROWS_PER_TILE)],
                    out_hbm.at[cid, pl.ds(sid * ROWS_PER_TILE, ROWS_PER_TILE)])
    pltpu.sync_copy(acc_den.at[pl.ds(sid * DEN_PER_TILE, DEN_PER_TILE)],
                    outden_hbm.at[cid, pl.ds(sid * DEN_PER_TILE, DEN_PER_TILE)])


_sc_params = pltpu.CompilerParams()
if "needs_layout_passes" in pltpu.CompilerParams.__dataclass_fields__:
    _sc_params = dataclasses.replace(_sc_params, needs_layout_passes=False)

_sc_edge_pass = functools.partial(
    pl.kernel,
    compiler_params=_sc_params,
    out_type=(jax.ShapeDtypeStruct((NC, N_PAD, DIM), jnp.float32),
              jax.ShapeDtypeStruct((NC, NDEN, DIM), jnp.float32)),
    mesh=plsc.VectorSubcoreMesh(core_axis_name="c", subcore_axis_name="s"),
    scratch_types=[
        pltpu.VMEM((BLK,), jnp.int32),
        pltpu.VMEM((BLK,), jnp.int32),
        pltpu.VMEM((BLK,), jnp.int32),
        pltpu.VMEM((BLK, DIM), jnp.float32),
        pltpu.VMEM((BLK, DIM), jnp.float32),
        pltpu.VMEM((BLK, HEADS), jnp.float32),
        pltpu.VMEM((BLK, DIM), jnp.float32),
        pltpu.VMEM_SHARED((N_PAD, DIM), jnp.float32),
        pltpu.VMEM_SHARED((NDEN, DIM), jnp.float32),
        pltpu.SemaphoreType.DMA,
        pltpu.SemaphoreType.DMA,
    ],
)(_sc_body)


def _final_body(n0_ref, n1_ref, d0_ref, d1_ref, wo_ref, bo_ref, o_ref):
    node = n0_ref[...] + n1_ref[...]
    den = d0_ref[...] + d1_ref[...]
    rr = (lax.broadcasted_iota(jnp.int32, (HEADS, DIM), 1) // DK
          == lax.broadcasted_iota(jnp.int32, (HEADS, DIM), 0)
          ).astype(jnp.float32)
    den_exp = lax.dot_general(den, rr, (((1,), (0,)), ((), ())),
                              precision=_HIGH,
                              preferred_element_type=jnp.float32)
    norm = node / (den_exp + 1e-16)
    o_ref[...] = _dotT(norm, wo_ref[...]) + bo_ref[...][None, :]


def _final(n0, n1, d0, d1, WO, bO):
    nb = 10
    blk = N // nb
    return pl.pallas_call(
        _final_body,
        grid=(nb,),
        in_specs=[
            pl.BlockSpec((blk, DIM), lambda i: (i, 0)),
            pl.BlockSpec((blk, DIM), lambda i: (i, 0)),
            pl.BlockSpec((blk, HEADS), lambda i: (i, 0)),
            pl.BlockSpec((blk, HEADS), lambda i: (i, 0)),
            pl.BlockSpec((DIM, DIM), lambda i: (0, 0)),
            pl.BlockSpec((DIM,), lambda i: (0,)),
        ],
        out_specs=pl.BlockSpec((blk, DIM), lambda i: (i, 0)),
        out_shape=jax.ShapeDtypeStruct((N, DIM), jnp.float32),
    )(n0, n1, d0, d1, WO, bO)


def kernel(x, edge_index, edge_attr, WQ, bQ, WK, bK, WV, bV, WO, bO,
           W1, b1, W2, b2):
    qs, ks, vs = _proj(x, WQ, bQ, WK, bK, WV, bV)
    bias = _edge_bias(edge_attr, W1, b1, W2, b2)
    i_idx = edge_index[0]
    j_idx = edge_index[1]
    node_p, den_p = _sc_edge_pass(qs, ks, vs, bias, i_idx, j_idx)
    den = den_p.reshape(NC, N_PAD, HEADS)
    n0 = node_p[0, :N]
    n1 = node_p[1, :N]
    d0 = den[0, :N]
    d1 = den[1, :N]
    return _final(n0, n1, d0, d1, WO, bO)


# speedup vs baseline: 1.6078x; 1.0502x over previous
import dataclasses
import functools
import math

import jax
import jax.numpy as jnp
from jax import lax
from jax.experimental import pallas as pl
from jax.experimental.pallas import tpu as pltpu
from jax.experimental.pallas import tpu_sc as plsc

N = 10000
E = 320000
DIM = 128
HEADS = 8
DK = DIM // HEADS
EDGE_DIM = 16

NC = 2
NS = 16
NW = NC * NS
EW = E // NW
BLK = 80
NBLK = EW // BLK
SUB = BLK // 16
N_PAD = 10240
ROWS_PER_TILE = N_PAD // NS
NDEN = N_PAD // 16
DEN_PER_TILE = NDEN // NS

_HIGH = jax.lax.Precision.HIGHEST


def _dotT(a, b):
    return lax.dot_general(a, b, (((1,), (1,)), ((), ())),
                           precision=_HIGH, preferred_element_type=jnp.float32)


def _proj_body(x_ref, wq_ref, bq_ref, wk_ref, bk_ref, wv_ref, bv_ref,
               q_ref, k_ref, v_ref):
    xb = x_ref[...]
    scale = 1.0 / math.sqrt(DK)
    q_ref[...] = (_dotT(xb, wq_ref[...]) + bq_ref[...][None, :]) * scale
    k_ref[...] = _dotT(xb, wk_ref[...]) + bk_ref[...][None, :]
    v_ref[...] = _dotT(xb, wv_ref[...]) + bv_ref[...][None, :]


def _proj(x, WQ, bQ, WK, bK, WV, bV):
    nb = 10
    blk = N // nb
    w_spec = pl.BlockSpec((DIM, DIM), lambda i: (0, 0))
    b_spec = pl.BlockSpec((DIM,), lambda i: (0,))
    row_spec = pl.BlockSpec((blk, DIM), lambda i: (i, 0))
    out = jax.ShapeDtypeStruct((N, DIM), jnp.float32)
    return pl.pallas_call(
        _proj_body,
        grid=(nb,),
        in_specs=[row_spec, w_spec, b_spec, w_spec, b_spec, w_spec, b_spec],
        out_specs=[row_spec, row_spec, row_spec],
        out_shape=[out, out, out],
    )(x, WQ, bQ, WK, bK, WV, bV)


def _bias_body(ea_ref, w1_ref, b1_ref, w2_ref, b2_ref, o_ref):
    h = _dotT(ea_ref[...], w1_ref[...]) + b1_ref[...][None, :]
    h = h * (1.0 / (1.0 + jnp.exp(-h)))
    o_ref[...] = _dotT(h, w2_ref[...]) + b2_ref[...][None, :]


def _edge_bias(edge_attr, W1, b1, W2, b2):
    nb = 80
    blk = E // nb
    return pl.pallas_call(
        _bias_body,
        grid=(nb,),
        in_specs=[
            pl.BlockSpec((blk, EDGE_DIM), lambda i: (i, 0)),
            pl.BlockSpec((EDGE_DIM, EDGE_DIM), lambda i: (0, 0)),
            pl.BlockSpec((EDGE_DIM,), lambda i: (0,)),
            pl.BlockSpec((HEADS, EDGE_DIM), lambda i: (0, 0)),
            pl.BlockSpec((HEADS,), lambda i: (0,)),
        ],
        out_specs=pl.BlockSpec((blk, HEADS), lambda i: (i, 0)),
        out_shape=jax.ShapeDtypeStruct((E, HEADS), jnp.float32),
    )(edge_attr, W1, b1, W2, b2)


def _sc_body(q_hbm, k_hbm, v_hbm, bias_hbm, i_hbm, j_hbm,
             out_hbm, outden_hbm,
             i_v, j_v, jdiv_v, q_v, k_v, bias_v, stage_den,
             acc, acc_den, sem0, sem1):
    cid = lax.axis_index("c")
    sid = lax.axis_index("s")
    wid = cid * NS + sid

    z16 = jnp.zeros((16,), jnp.float32)

    @pl.loop(0, BLK)
    def _(r):
        for c in range(DIM // 16):
            stage_den[r, pl.ds(c * 16, 16)] = z16

    @pl.loop(0, ROWS_PER_TILE // BLK)
    def _(b):
        pltpu.sync_copy(stage_den,
                        acc.at[pl.ds(sid * ROWS_PER_TILE + b * BLK, BLK)])

    pltpu.sync_copy(stage_den.at[pl.ds(0, DEN_PER_TILE)],
                    acc_den.at[pl.ds(sid * DEN_PER_TILE, DEN_PER_TILE)])

    plsc.subcore_barrier()

    wbase = wid * EW

    @pl.loop(0, NBLK)
    def _(blk):
        base = wbase + blk * BLK
        pltpu.sync_copy(i_hbm.at[pl.ds(base, BLK)], i_v)
        pltpu.sync_copy(j_hbm.at[pl.ds(base, BLK)], j_v)
        pltpu.sync_copy(bias_hbm.at[pl.ds(base, BLK)], bias_v)
        cq = pltpu.async_copy(q_hbm.at[j_v], q_v, sem0)
        ck = pltpu.async_copy(k_hbm.at[i_v], k_v, sem1)

        @pl.loop(0, SUB)
        def _(sb):
            sl = pl.ds(sb * 16, 16)
            jdiv_v[sl] = lax.shift_right_logical(j_v[sl], 4)

        cq.wait()
        ck.wait()

        @plsc.parallel_loop(0, SUB, unroll=2)
        def _(sb):
            eidx = sb * 16 + lax.iota(jnp.int32, 16)
            j16 = plsc.load_gather(j_v, [eidx])
            colbase = lax.shift_left(jnp.bitwise_and(j16, 15), 3)
            for h in range(HEADS):
                part = [z16, z16, z16, z16]
                for dd in range(DK):
                    col = jnp.full((16,), h * DK + dd, jnp.int32)
                    qv = plsc.load_gather(q_v, [eidx, col])
                    kv = plsc.load_gather(k_v, [eidx, col])
                    part[dd % 4] = part[dd % 4] + qv * kv
                acc16 = (part[0] + part[1]) + (part[2] + part[3])
                hcol = jnp.full((16,), h, jnp.int32)
                bv = plsc.load_gather(bias_v, [eidx, hcol])
                ex = jnp.exp(acc16 + bv)
                plsc.store_scatter(bias_v, [eidx, hcol], ex)
                plsc.store_scatter(stage_den, [eidx, colbase + h], ex)

        cv = pltpu.async_copy(v_hbm.at[i_v], q_v, sem0)
        pltpu.sync_copy(stage_den, acc_den.at[jdiv_v], add=True)
        cv.wait()

        @plsc.parallel_loop(0, SUB, unroll=2)
        def _(sb):
            eidx = sb * 16 + lax.iota(jnp.int32, 16)
            for h in range(HEADS):
                ex = plsc.load_gather(bias_v, [eidx, jnp.full((16,), h, jnp.int32)])
                for dd in range(DK):
                    col = jnp.full((16,), h * DK + dd, jnp.int32)
                    vv = plsc.load_gather(q_v, [eidx, col])
                    plsc.store_scatter(k_v, [eidx, col], ex * vv)

        pltpu.sync_copy(k_v, acc.at[j_v], add=True)

        @plsc.parallel_loop(0, SUB, unroll=2)
        def _(sb):
            eidx = sb * 16 + lax.iota(jnp.int32, 16)
            j16 = plsc.load_gather(j_v, [eidx])
            colbase = lax.shift_left(jnp.bitwise_and(j16, 15), 3)
            for h in range(HEADS):
                plsc.store_scatter(stage_den, [eidx, colbase + h], z16)

    plsc.subcore_barrier()

    pltpu.sync_copy(acc.at[pl.ds(sid * ROWS_PER_TILE, ROWS_PER_TILE)],
                    out_hbm.at[cid, pl.ds(sid * ROWS_PER_TILE, ROWS_PER_TILE)])
    pltpu.sync_copy(acc_den.at[pl.ds(sid * DEN_PER_TILE, DEN_PER_TILE)],
                    outden_hbm.at[cid, pl.ds(sid * DEN_PER_TILE, DEN_PER_TILE)])


_sc_params = pltpu.CompilerParams()
if "needs_layout_passes" in pltpu.CompilerParams.__dataclass_fields__:
    _sc_params = dataclasses.replace(_sc_params, needs_layout_passes=False)

_sc_edge_pass = functools.partial(
    pl.kernel,
    compiler_params=_sc_params,
    out_type=(jax.ShapeDtypeStruct((NC, N_PAD, DIM), jnp.float32),
              jax.ShapeDtypeStruct((NC, NDEN, DIM), jnp.float32)),
    mesh=plsc.VectorSubcoreMesh(core_axis_name="c", subcore_axis_name="s"),
    scratch_types=[
        pltpu.VMEM((BLK,), jnp.int32),
        pltpu.VMEM((BLK,), jnp.int32),
        pltpu.VMEM((BLK,), jnp.int32),
        pltpu.VMEM((BLK, DIM), jnp.float32),
        pltpu.VMEM((BLK, DIM), jnp.float32),
        pltpu.VMEM((BLK, HEADS), jnp.float32),
        pltpu.VMEM((BLK, DIM), jnp.float32),
        pltpu.VMEM_SHARED((N_PAD, DIM), jnp.float32),
        pltpu.VMEM_SHARED((NDEN, DIM), jnp.float32),
        pltpu.SemaphoreType.DMA,
        pltpu.SemaphoreType.DMA,
    ],
)(_sc_body)


def _final_body(n0_ref, n1_ref, d0_ref, d1_ref, wo_ref, bo_ref, o_ref):
    node = n0_ref[...] + n1_ref[...]
    den = d0_ref[...] + d1_ref[...]
    rr = (lax.broadcasted_iota(jnp.int32, (HEADS, DIM), 1) // DK
          == lax.broadcasted_iota(jnp.int32, (HEADS, DIM), 0)
          ).astype(jnp.float32)
    den_exp = lax.dot_general(den, rr, (((1,), (0,)), ((), ())),
                              precision=_HIGH,
                              preferred_element_type=jnp.float32)
    norm = node / (den_exp + 1e-16)
    o_ref[...] = _dotT(norm, wo_ref[...]) + bo_ref[...][None, :]


def _final(n0, n1, d0, d1, WO, bO):
    nb = 10
    blk = N // nb
    return pl.pallas_call(
        _final_body,
        grid=(nb,),
        in_specs=[
            pl.BlockSpec((blk, DIM), lambda i: (i, 0)),
            pl.BlockSpec((blk, DIM), lambda i: (i, 0)),
            pl.BlockSpec((blk, HEADS), lambda i: (i, 0)),
            pl.BlockSpec((blk, HEADS), lambda i: (i, 0)),
            pl.BlockSpec((DIM, DIM), lambda i: (0, 0)),
            pl.BlockSpec((DIM,), lambda i: (0,)),
        ],
        out_specs=pl.BlockSpec((blk, DIM), lambda i: (i, 0)),
        out_shape=jax.ShapeDtypeStruct((N, DIM), jnp.float32),
    )(n0, n1, d0, d1, WO, bO)


def kernel(x, edge_index, edge_attr, WQ, bQ, WK, bK, WV, bV, WO, bO,
           W1, b1, W2, b2):
    qs, ks, vs = _proj(x, WQ, bQ, WK, bK, WV, bV)
    bias = _edge_bias(edge_attr, W1, b1, W2, b2)
    i_idx = edge_index[0]
    j_idx = edge_index[1]
    node_p, den_p = _sc_edge_pass(qs, ks, vs, bias, i_idx, j_idx)
    den = den_p.reshape(NC, N_PAD, HEADS)
    n0 = node_p[0, :N]
    n1 = node_p[1, :N]
    d0 = den[0, :N]
    d1 = den[1, :N]
    return _final(n0, n1, d0, d1, WO, bO)


# speedup vs baseline: 4.0888x; 2.5430x over previous
import dataclasses
import functools
import math

import jax
import jax.numpy as jnp
from jax import lax
from jax.experimental import pallas as pl
from jax.experimental.pallas import tpu as pltpu
from jax.experimental.pallas import tpu_sc as plsc

N = 10000
E = 320000
DIM = 128
HEADS = 8
DK = DIM // HEADS
EDGE_DIM = 16

NC = 2
NS = 16
NW = NC * NS
EW = E // NW
BLK = 80
NBLK = EW // BLK
SUB = BLK // 16
N_PAD = 10240
ROWS_PER_TILE = N_PAD // NS
NDEN = N_PAD // 16
DEN_PER_TILE = NDEN // NS

_HIGH = jax.lax.Precision.HIGHEST


def _dotT(a, b):
    return lax.dot_general(a, b, (((1,), (1,)), ((), ())),
                           precision=_HIGH, preferred_element_type=jnp.float32)


def _proj_body(x_ref, wq_ref, bq_ref, wk_ref, bk_ref, wv_ref, bv_ref,
               q_ref, k_ref, v_ref):
    xb = x_ref[...]
    scale = 1.0 / math.sqrt(DK)
    q_ref[...] = (_dotT(xb, wq_ref[...]) + bq_ref[...][None, :]) * scale
    k_ref[...] = _dotT(xb, wk_ref[...]) + bk_ref[...][None, :]
    v_ref[...] = _dotT(xb, wv_ref[...]) + bv_ref[...][None, :]


def _proj(x, WQ, bQ, WK, bK, WV, bV):
    nb = 10
    blk = N // nb
    w_spec = pl.BlockSpec((DIM, DIM), lambda i: (0, 0))
    b_spec = pl.BlockSpec((DIM,), lambda i: (0,))
    row_spec = pl.BlockSpec((blk, DIM), lambda i: (i, 0))
    out = jax.ShapeDtypeStruct((N, DIM), jnp.float32)
    return pl.pallas_call(
        _proj_body,
        grid=(nb,),
        in_specs=[row_spec, w_spec, b_spec, w_spec, b_spec, w_spec, b_spec],
        out_specs=[row_spec, row_spec, row_spec],
        out_shape=[out, out, out],
    )(x, WQ, bQ, WK, bK, WV, bV)


def _bias_body(ea_ref, w1_ref, b1_ref, w2_ref, b2_ref, o_ref):
    h = _dotT(ea_ref[...], w1_ref[...]) + b1_ref[...][None, :]
    h = h * (1.0 / (1.0 + jnp.exp(-h)))
    o_ref[...] = _dotT(h, w2_ref[...]) + b2_ref[...][None, :]


def _edge_bias(edge_attr, W1, b1, W2, b2):
    nb = 80
    blk = E // nb
    return pl.pallas_call(
        _bias_body,
        grid=(nb,),
        in_specs=[
            pl.BlockSpec((blk, EDGE_DIM), lambda i: (i, 0)),
            pl.BlockSpec((EDGE_DIM, EDGE_DIM), lambda i: (0, 0)),
            pl.BlockSpec((EDGE_DIM,), lambda i: (0,)),
            pl.BlockSpec((HEADS, EDGE_DIM), lambda i: (0, 0)),
            pl.BlockSpec((HEADS,), lambda i: (0,)),
        ],
        out_specs=pl.BlockSpec((blk, HEADS), lambda i: (i, 0)),
        out_shape=jax.ShapeDtypeStruct((E, HEADS), jnp.float32),
    )(edge_attr, W1, b1, W2, b2)


def _sc_body(q_hbm, k_hbm, v_hbm, bias_hbm, i_hbm, j_hbm,
             out_hbm, outden_hbm,
             i_v, j_v, jdiv_v, q_v, k_v, bias_v, score_v, stage_den,
             acc, acc_den, sem0, sem1):
    cid = lax.axis_index("c")
    sid = lax.axis_index("s")
    wid = cid * NS + sid

    z16 = jnp.zeros((16,), jnp.float32)
    lane = lax.iota(jnp.int32, 16)
    lane15 = lane == 15
    hi8 = jnp.where(lane >= 8, 1, 0)
    lane7 = jnp.bitwise_and(lane, 7)

    @pl.loop(0, BLK)
    def _(r):
        for c in range(DIM // 16):
            stage_den[r, pl.ds(c * 16, 16)] = z16

    @pl.loop(0, ROWS_PER_TILE // BLK)
    def _(b):
        pltpu.sync_copy(stage_den,
                        acc.at[pl.ds(sid * ROWS_PER_TILE + b * BLK, BLK)])

    pltpu.sync_copy(stage_den.at[pl.ds(0, DEN_PER_TILE)],
                    acc_den.at[pl.ds(sid * DEN_PER_TILE, DEN_PER_TILE)])

    plsc.subcore_barrier()

    wbase = wid * EW

    @pl.loop(0, NBLK)
    def _(blk):
        base = wbase + blk * BLK
        pltpu.sync_copy(i_hbm.at[pl.ds(base, BLK)], i_v)
        pltpu.sync_copy(j_hbm.at[pl.ds(base, BLK)], j_v)
        pltpu.sync_copy(bias_hbm.at[pl.ds(base * HEADS, BLK * HEADS)], bias_v)
        cq = pltpu.async_copy(q_hbm.at[j_v], q_v, sem0)
        ck = pltpu.async_copy(k_hbm.at[i_v], k_v, sem1)

        @pl.loop(0, SUB)
        def _(sb):
            sl = pl.ds(sb * 16, 16)
            jdiv_v[sl] = lax.shift_right_logical(j_v[sl], 4)

        cq.wait()
        ck.wait()

        @plsc.parallel_loop(0, BLK, unroll=2)
        def _(e):
            for h in range(HEADS):
                prod = (q_v[e, pl.ds(h * DK, 16)] * k_v[e, pl.ds(h * DK, 16)])
                cums = plsc.cumsum(prod)
                plsc.store_scatter(score_v, [jnp.full((16,), e * 8 + h, jnp.int32)],
                                   cums, mask=lane15)

        cv = pltpu.async_copy(v_hbm.at[i_v], q_v, sem0)

        @plsc.parallel_loop(0, BLK * HEADS // 16, unroll=2)
        def _(c):
            sl = pl.ds(c * 16, 16)
            bias_v[sl] = jnp.exp(score_v[sl] + bias_v[sl])

        @plsc.parallel_loop(0, BLK // 2, unroll=2)
        def _(p):
            rowv = 2 * p + hi8
            jp = plsc.load_gather(j_v, [rowv])
            colv = lax.shift_left(jnp.bitwise_and(jp, 15), 3) + lane7
            exv = bias_v[pl.ds(p * 16, 16)]
            plsc.store_scatter(stage_den, [rowv, colv], exv)

        pltpu.sync_copy(stage_den, acc_den.at[jdiv_v], add=True)
        cv.wait()

        @plsc.parallel_loop(0, BLK // 2, unroll=2)
        def _(p):
            exv = bias_v[pl.ds(p * 16, 16)]
            for sub_e in range(2):
                e = 2 * p + sub_e
                for h in range(HEADS):
                    bc = exv.at[jnp.full((16,), sub_e * 8 + h, jnp.int32)].get(
                        mode="promise_in_bounds")
                    k_v[e, pl.ds(h * DK, 16)] = bc * q_v[e, pl.ds(h * DK, 16)]

        pltpu.sync_copy(k_v, acc.at[j_v], add=True)

        @plsc.parallel_loop(0, BLK // 2, unroll=2)
        def _(p):
            rowv = 2 * p + hi8
            jp = plsc.load_gather(j_v, [rowv])
            colv = lax.shift_left(jnp.bitwise_and(jp, 15), 3) + lane7
            plsc.store_scatter(stage_den, [rowv, colv], z16)

    plsc.subcore_barrier()

    pltpu.sync_copy(acc.at[pl.ds(sid * ROWS_PER_TILE, ROWS_PER_TILE)],
                    out_hbm.at[cid, pl.ds(sid * ROWS_PER_TILE, ROWS_PER_TILE)])
    pltpu.sync_copy(acc_den.at[pl.ds(sid * DEN_PER_TILE, DEN_PER_TILE)],
                    outden_hbm.at[cid, pl.ds(sid * DEN_PER_TILE, DEN_PER_TILE)])


_sc_params = pltpu.CompilerParams()
if "needs_layout_passes" in pltpu.CompilerParams.__dataclass_fields__:
    _sc_params = dataclasses.replace(_sc_params, needs_layout_passes=False)

_sc_edge_pass = functools.partial(
    pl.kernel,
    compiler_params=_sc_params,
    out_type=(jax.ShapeDtypeStruct((NC, N_PAD, DIM), jnp.float32),
              jax.ShapeDtypeStruct((NC, NDEN, DIM), jnp.float32)),
    mesh=plsc.VectorSubcoreMesh(core_axis_name="c", subcore_axis_name="s"),
    scratch_types=[
        pltpu.VMEM((BLK,), jnp.int32),
        pltpu.VMEM((BLK,), jnp.int32),
        pltpu.VMEM((BLK,), jnp.int32),
        pltpu.VMEM((BLK, DIM), jnp.float32),
        pltpu.VMEM((BLK, DIM), jnp.float32),
        pltpu.VMEM((BLK * HEADS,), jnp.float32),
        pltpu.VMEM((BLK * HEADS,), jnp.float32),
        pltpu.VMEM((BLK, DIM), jnp.float32),
        pltpu.VMEM_SHARED((N_PAD, DIM), jnp.float32),
        pltpu.VMEM_SHARED((NDEN, DIM), jnp.float32),
        pltpu.SemaphoreType.DMA,
        pltpu.SemaphoreType.DMA,
    ],
)(_sc_body)


def _final_body(n0_ref, n1_ref, d0_ref, d1_ref, wo_ref, bo_ref, o_ref):
    node = n0_ref[...] + n1_ref[...]
    den = d0_ref[...] + d1_ref[...]
    rr = (lax.broadcasted_iota(jnp.int32, (HEADS, DIM), 1) // DK
          == lax.broadcasted_iota(jnp.int32, (HEADS, DIM), 0)
          ).astype(jnp.float32)
    den_exp = lax.dot_general(den, rr, (((1,), (0,)), ((), ())),
                              precision=_HIGH,
                              preferred_element_type=jnp.float32)
    norm = node / (den_exp + 1e-16)
    o_ref[...] = _dotT(norm, wo_ref[...]) + bo_ref[...][None, :]


def _final(n0, n1, d0, d1, WO, bO):
    nb = 10
    blk = N // nb
    return pl.pallas_call(
        _final_body,
        grid=(nb,),
        in_specs=[
            pl.BlockSpec((blk, DIM), lambda i: (i, 0)),
            pl.BlockSpec((blk, DIM), lambda i: (i, 0)),
            pl.BlockSpec((blk, HEADS), lambda i: (i, 0)),
            pl.BlockSpec((blk, HEADS), lambda i: (i, 0)),
            pl.BlockSpec((DIM, DIM), lambda i: (0, 0)),
            pl.BlockSpec((DIM,), lambda i: (0,)),
        ],
        out_specs=pl.BlockSpec((blk, DIM), lambda i: (i, 0)),
        out_shape=jax.ShapeDtypeStruct((N, DIM), jnp.float32),
    )(n0, n1, d0, d1, WO, bO)


def kernel(x, edge_index, edge_attr, WQ, bQ, WK, bK, WV, bV, WO, bO,
           W1, b1, W2, b2):
    qs, ks, vs = _proj(x, WQ, bQ, WK, bK, WV, bV)
    bias = _edge_bias(edge_attr, W1, b1, W2, b2)
    i_idx = edge_index[0]
    j_idx = edge_index[1]
    node_p, den_p = _sc_edge_pass(qs, ks, vs, bias.reshape(E * HEADS), i_idx,
                                  j_idx)
    den = den_p.reshape(NC, N_PAD, HEADS)
    n0 = node_p[0, :N]
    n1 = node_p[1, :N]
    d0 = den[0, :N]
    d1 = den[1, :N]
    return _final(n0, n1, d0, d1, WO, bO)


# speedup vs baseline: 6.2881x; 1.5379x over previous
import dataclasses
import functools
import math

import jax
import jax.numpy as jnp
from jax import lax
from jax.experimental import pallas as pl
from jax.experimental.pallas import tpu as pltpu
from jax.experimental.pallas import tpu_sc as plsc

N = 10000
E = 320000
DIM = 128
HEADS = 8
DK = DIM // HEADS
EDGE_DIM = 16

NC = 2
NS = 16
NW = NC * NS
EW = E // NW
BLK = 80
NBLK = EW // BLK
SUB = BLK // 16
N_PAD = 10240
ROWS_PER_TILE = N_PAD // NS
NDEN = N_PAD // 16
DEN_PER_TILE = NDEN // NS

_HIGH = jax.lax.Precision.HIGHEST


def _dotT(a, b):
    return lax.dot_general(a, b, (((1,), (1,)), ((), ())),
                           precision=_HIGH, preferred_element_type=jnp.float32)


def _proj_body(x_ref, wq_ref, bq_ref, wk_ref, bk_ref, wv_ref, bv_ref,
               q_ref, k_ref, v_ref):
    xb = x_ref[...]
    scale = 1.0 / math.sqrt(DK)
    q_ref[...] = (_dotT(xb, wq_ref[...]) + bq_ref[...][None, :]) * scale
    k_ref[...] = _dotT(xb, wk_ref[...]) + bk_ref[...][None, :]
    v_ref[...] = _dotT(xb, wv_ref[...]) + bv_ref[...][None, :]


def _proj(x, WQ, bQ, WK, bK, WV, bV):
    nb = 10
    blk = N // nb
    w_spec = pl.BlockSpec((DIM, DIM), lambda i: (0, 0))
    b_spec = pl.BlockSpec((DIM,), lambda i: (0,))
    row_spec = pl.BlockSpec((blk, DIM), lambda i: (i, 0))
    out = jax.ShapeDtypeStruct((N, DIM), jnp.float32)
    return pl.pallas_call(
        _proj_body,
        grid=(nb,),
        in_specs=[row_spec, w_spec, b_spec, w_spec, b_spec, w_spec, b_spec],
        out_specs=[row_spec, row_spec, row_spec],
        out_shape=[out, out, out],
    )(x, WQ, bQ, WK, bK, WV, bV)


def _bias_body(ea_ref, w1_ref, b1_ref, w2_ref, b2_ref, o_ref):
    h = (lax.dot_general(ea_ref[...], w1_ref[...], (((1,), (0,)), ((), ())),
                         precision=_HIGH, preferred_element_type=jnp.float32)
         + b1_ref[...][None, :])
    h = h * (1.0 / (1.0 + jnp.exp(-h)))
    o_ref[...] = (lax.dot_general(h, w2_ref[...], (((1,), (0,)), ((), ())),
                                  precision=_HIGH,
                                  preferred_element_type=jnp.float32)
                  + b2_ref[...][None, :])


def _edge_bias(edge_attr, W1, b1, W2, b2):
    ep = E // 8
    ea_p = edge_attr.reshape(ep, 8 * EDGE_DIM)
    w1bd = jnp.kron(jnp.eye(8, dtype=jnp.float32), W1.T)
    b1t = jnp.tile(b1, 8)
    w2bd = jnp.kron(jnp.eye(8, dtype=jnp.float32), W2.T)
    b2t = jnp.tile(b2, 8)
    nb = 10
    blk = ep // nb
    out = pl.pallas_call(
        _bias_body,
        grid=(nb,),
        in_specs=[
            pl.BlockSpec((blk, 8 * EDGE_DIM), lambda i: (i, 0)),
            pl.BlockSpec((8 * EDGE_DIM, 8 * EDGE_DIM), lambda i: (0, 0)),
            pl.BlockSpec((8 * EDGE_DIM,), lambda i: (0,)),
            pl.BlockSpec((8 * EDGE_DIM, 8 * HEADS), lambda i: (0, 0)),
            pl.BlockSpec((8 * HEADS,), lambda i: (0,)),
        ],
        out_specs=pl.BlockSpec((blk, 8 * HEADS), lambda i: (i, 0)),
        out_shape=jax.ShapeDtypeStruct((ep, 8 * HEADS), jnp.float32),
    )(ea_p, w1bd, b1t, w2bd, b2t)
    return out.reshape(E * HEADS)


def _sc_body(q_hbm, k_hbm, v_hbm, bias_hbm, i_hbm, j_hbm,
             out_hbm, outden_hbm,
             i_v, j_v, jdiv_v, q_v, k_v, bias_v, score_v, stage_den,
             acc, acc_den, sem0, sem1):
    cid = lax.axis_index("c")
    sid = lax.axis_index("s")
    wid = cid * NS + sid

    z16 = jnp.zeros((16,), jnp.float32)
    lane = lax.iota(jnp.int32, 16)
    lane15 = lane == 15
    hi8 = jnp.where(lane >= 8, 1, 0)
    lane7 = jnp.bitwise_and(lane, 7)

    @pl.loop(0, BLK)
    def _(r):
        for c in range(DIM // 16):
            stage_den[r, pl.ds(c * 16, 16)] = z16

    @pl.loop(0, ROWS_PER_TILE // BLK)
    def _(b):
        pltpu.sync_copy(stage_den,
                        acc.at[pl.ds(sid * ROWS_PER_TILE + b * BLK, BLK)])

    pltpu.sync_copy(stage_den.at[pl.ds(0, DEN_PER_TILE)],
                    acc_den.at[pl.ds(sid * DEN_PER_TILE, DEN_PER_TILE)])

    plsc.subcore_barrier()

    wbase = wid * EW

    @pl.loop(0, NBLK)
    def _(blk):
        base = wbase + blk * BLK
        pltpu.sync_copy(i_hbm.at[pl.ds(base, BLK)], i_v)
        pltpu.sync_copy(j_hbm.at[pl.ds(base, BLK)], j_v)
        pltpu.sync_copy(bias_hbm.at[pl.ds(base * HEADS, BLK * HEADS)], bias_v)
        cq = pltpu.async_copy(q_hbm.at[j_v], q_v, sem0)
        ck = pltpu.async_copy(k_hbm.at[i_v], k_v, sem1)

        @pl.loop(0, SUB)
        def _(sb):
            sl = pl.ds(sb * 16, 16)
            jdiv_v[sl] = lax.shift_right_logical(j_v[sl], 4)

        cq.wait()
        ck.wait()

        @plsc.parallel_loop(0, BLK, unroll=2)
        def _(e):
            for h in range(HEADS):
                prod = (q_v[e, pl.ds(h * DK, 16)] * k_v[e, pl.ds(h * DK, 16)])
                cums = plsc.cumsum(prod)
                plsc.store_scatter(score_v, [jnp.full((16,), e * 8 + h, jnp.int32)],
                                   cums, mask=lane15)

        cv = pltpu.async_copy(v_hbm.at[i_v], q_v, sem0)

        @plsc.parallel_loop(0, BLK * HEADS // 16, unroll=2)
        def _(c):
            sl = pl.ds(c * 16, 16)
            bias_v[sl] = jnp.exp(score_v[sl] + bias_v[sl])

        @plsc.parallel_loop(0, BLK // 2, unroll=2)
        def _(p):
            rowv = 2 * p + hi8
            jp = plsc.load_gather(j_v, [rowv])
            colv = lax.shift_left(jnp.bitwise_and(jp, 15), 3) + lane7
            exv = bias_v[pl.ds(p * 16, 16)]
            plsc.store_scatter(stage_den, [rowv, colv], exv)

        pltpu.sync_copy(stage_den, acc_den.at[jdiv_v], add=True)
        cv.wait()

        @plsc.parallel_loop(0, BLK // 2, unroll=2)
        def _(p):
            exv = bias_v[pl.ds(p * 16, 16)]
            for sub_e in range(2):
                e = 2 * p + sub_e
                for h in range(HEADS):
                    bc = exv.at[jnp.full((16,), sub_e * 8 + h, jnp.int32)].get(
                        mode="promise_in_bounds")
                    k_v[e, pl.ds(h * DK, 16)] = bc * q_v[e, pl.ds(h * DK, 16)]

        pltpu.sync_copy(k_v, acc.at[j_v], add=True)

        @plsc.parallel_loop(0, BLK // 2, unroll=2)
        def _(p):
            rowv = 2 * p + hi8
            jp = plsc.load_gather(j_v, [rowv])
            colv = lax.shift_left(jnp.bitwise_and(jp, 15), 3) + lane7
            plsc.store_scatter(stage_den, [rowv, colv], z16)

    plsc.subcore_barrier()

    pltpu.sync_copy(acc.at[pl.ds(sid * ROWS_PER_TILE, ROWS_PER_TILE)],
                    out_hbm.at[cid, pl.ds(sid * ROWS_PER_TILE, ROWS_PER_TILE)])
    pltpu.sync_copy(acc_den.at[pl.ds(sid * DEN_PER_TILE, DEN_PER_TILE)],
                    outden_hbm.at[cid, pl.ds(sid * DEN_PER_TILE, DEN_PER_TILE)])


_sc_params = pltpu.CompilerParams()
if "needs_layout_passes" in pltpu.CompilerParams.__dataclass_fields__:
    _sc_params = dataclasses.replace(_sc_params, needs_layout_passes=False)

_sc_edge_pass = functools.partial(
    pl.kernel,
    compiler_params=_sc_params,
    out_type=(jax.ShapeDtypeStruct((NC, N_PAD, DIM), jnp.float32),
              jax.ShapeDtypeStruct((NC, NDEN, DIM), jnp.float32)),
    mesh=plsc.VectorSubcoreMesh(core_axis_name="c", subcore_axis_name="s"),
    scratch_types=[
        pltpu.VMEM((BLK,), jnp.int32),
        pltpu.VMEM((BLK,), jnp.int32),
        pltpu.VMEM((BLK,), jnp.int32),
        pltpu.VMEM((BLK, DIM), jnp.float32),
        pltpu.VMEM((BLK, DIM), jnp.float32),
        pltpu.VMEM((BLK * HEADS,), jnp.float32),
        pltpu.VMEM((BLK * HEADS,), jnp.float32),
        pltpu.VMEM((BLK, DIM), jnp.float32),
        pltpu.VMEM_SHARED((N_PAD, DIM), jnp.float32),
        pltpu.VMEM_SHARED((NDEN, DIM), jnp.float32),
        pltpu.SemaphoreType.DMA,
        pltpu.SemaphoreType.DMA,
    ],
)(_sc_body)


def _final_body(np_ref, dp_ref, wo_ref, bo_ref, o_ref):
    node = np_ref[0, :N] + np_ref[1, :N]
    den = dp_ref[0] + dp_ref[1]
    k_i = lax.broadcasted_iota(jnp.int32, (DIM, 16 * DIM), 0)
    q_i = lax.broadcasted_iota(jnp.int32, (DIM, 16 * DIM), 1)
    mm = (k_i == 8 * (q_i // DIM) + (q_i % DIM) // DK).astype(jnp.float32)
    den_exp = lax.dot_general(den, mm, (((1,), (0,)), ((), ())),
                              precision=_HIGH,
                              preferred_element_type=jnp.float32)
    den_exp = den_exp.reshape(N_PAD, DIM)[:N]
    norm = node / (den_exp + 1e-16)
    o_ref[...] = _dotT(norm, wo_ref[...]) + bo_ref[...][None, :]


def _final(node_p, den_p, WO, bO):
    return pl.pallas_call(
        _final_body,
        grid=(1,),
        in_specs=[
            pl.BlockSpec((NC, N_PAD, DIM), lambda i: (0, 0, 0)),
            pl.BlockSpec((NC, NDEN, DIM), lambda i: (0, 0, 0)),
            pl.BlockSpec((DIM, DIM), lambda i: (0, 0)),
            pl.BlockSpec((DIM,), lambda i: (0,)),
        ],
        out_specs=pl.BlockSpec((N, DIM), lambda i: (0, 0)),
        out_shape=jax.ShapeDtypeStruct((N, DIM), jnp.float32),
    )(node_p, den_p, WO, bO)


def kernel(x, edge_index, edge_attr, WQ, bQ, WK, bK, WV, bV, WO, bO,
           W1, b1, W2, b2):
    qs, ks, vs = _proj(x, WQ, bQ, WK, bK, WV, bV)
    bias = _edge_bias(edge_attr, W1, b1, W2, b2)
    i_idx = edge_index[0]
    j_idx = edge_index[1]
    node_p, den_p = _sc_edge_pass(qs, ks, vs, bias, i_idx, j_idx)
    return _final(node_p, den_p, WO, bO)


# speedup vs baseline: 7.7451x; 1.2317x over previous
import dataclasses
import functools
import math

import jax
import jax.numpy as jnp
from jax import lax
from jax.experimental import pallas as pl
from jax.experimental.pallas import tpu as pltpu
from jax.experimental.pallas import tpu_sc as plsc

N = 10000
E = 320000
DIM = 128
HEADS = 8
DK = DIM // HEADS
EDGE_DIM = 16

NC = 2
NS = 16
NW = NC * NS
EW = E // NW
BLK = 80
NBLK = EW // BLK
SUB = BLK // 16
N_PAD = 10240
ROWS_PER_TILE = N_PAD // NS
NDEN = N_PAD // 16
DEN_PER_TILE = NDEN // NS

_HIGH = jax.lax.Precision.HIGHEST


def _dotT(a, b):
    return lax.dot_general(a, b, (((1,), (1,)), ((), ())),
                           precision=_HIGH, preferred_element_type=jnp.float32)


def _proj_body(x_ref, wq_ref, bq_ref, wk_ref, bk_ref, wv_ref, bv_ref,
               q_ref, k_ref, v_ref):
    xb = x_ref[...]
    scale = 1.0 / math.sqrt(DK)
    q_ref[...] = (_dotT(xb, wq_ref[...]) + bq_ref[...][None, :]) * scale
    k_ref[...] = _dotT(xb, wk_ref[...]) + bk_ref[...][None, :]
    v_ref[...] = _dotT(xb, wv_ref[...]) + bv_ref[...][None, :]


def _proj(x, WQ, bQ, WK, bK, WV, bV):
    nb = 10
    blk = N // nb
    w_spec = pl.BlockSpec((DIM, DIM), lambda i: (0, 0))
    b_spec = pl.BlockSpec((DIM,), lambda i: (0,))
    row_spec = pl.BlockSpec((blk, DIM), lambda i: (i, 0))
    out = jax.ShapeDtypeStruct((N, DIM), jnp.float32)
    return pl.pallas_call(
        _proj_body,
        grid=(nb,),
        in_specs=[row_spec, w_spec, b_spec, w_spec, b_spec, w_spec, b_spec],
        out_specs=[row_spec, row_spec, row_spec],
        out_shape=[out, out, out],
    )(x, WQ, bQ, WK, bK, WV, bV)


def _bias_body(ea_ref, w1_ref, b1_ref, w2_ref, b2_ref, o_ref):
    h = (lax.dot_general(ea_ref[...], w1_ref[...], (((1,), (0,)), ((), ())),
                         precision=_HIGH, preferred_element_type=jnp.float32)
         + b1_ref[...][None, :])
    h = h * (1.0 / (1.0 + jnp.exp(-h)))
    o_ref[...] = (lax.dot_general(h, w2_ref[...], (((1,), (0,)), ((), ())),
                                  precision=_HIGH,
                                  preferred_element_type=jnp.float32)
                  + b2_ref[...][None, :])


def _edge_bias(edge_attr, W1, b1, W2, b2):
    ep = E // 8
    ea_p = edge_attr.reshape(ep, 8 * EDGE_DIM)
    w1bd = jnp.kron(jnp.eye(8, dtype=jnp.float32), W1.T)
    b1t = jnp.tile(b1, 8)
    w2bd = jnp.kron(jnp.eye(8, dtype=jnp.float32), W2.T)
    b2t = jnp.tile(b2, 8)
    nb = 10
    blk = ep // nb
    out = pl.pallas_call(
        _bias_body,
        grid=(nb,),
        in_specs=[
            pl.BlockSpec((blk, 8 * EDGE_DIM), lambda i: (i, 0)),
            pl.BlockSpec((8 * EDGE_DIM, 8 * EDGE_DIM), lambda i: (0, 0)),
            pl.BlockSpec((8 * EDGE_DIM,), lambda i: (0,)),
            pl.BlockSpec((8 * EDGE_DIM, 8 * HEADS), lambda i: (0, 0)),
            pl.BlockSpec((8 * HEADS,), lambda i: (0,)),
        ],
        out_specs=pl.BlockSpec((blk, 8 * HEADS), lambda i: (i, 0)),
        out_shape=jax.ShapeDtypeStruct((ep, 8 * HEADS), jnp.float32),
    )(ea_p, w1bd, b1t, w2bd, b2t)
    return out.reshape(E * HEADS)


def _sc_body(q_hbm, k_hbm, v_hbm, bias_hbm, i_hbm, j_hbm,
             out_hbm, outden_hbm,
             i_v0, i_v1, j_v0, j_v1, jdiv_v0, jdiv_v1, bias_v0, bias_v1,
             score_v, q_v, k_v, stage_den,
             acc, acc_den, semq, semk, semv, semvs, semds, seml):
    cid = lax.axis_index("c")
    sid = lax.axis_index("s")
    wid = cid * NS + sid

    z16 = jnp.zeros((16,), jnp.float32)
    lane = lax.iota(jnp.int32, 16)
    lane15 = lane == 15
    hi8 = jnp.where(lane >= 8, 1, 0)
    lane7 = jnp.bitwise_and(lane, 7)

    i_b = (i_v0, i_v1)
    j_b = (j_v0, j_v1)
    jdiv_b = (jdiv_v0, jdiv_v1)
    bias_b = (bias_v0, bias_v1)

    @pl.loop(0, BLK)
    def _(r):
        for c in range(DIM // 16):
            stage_den[r, pl.ds(c * 16, 16)] = z16

    @pl.loop(0, ROWS_PER_TILE // BLK)
    def _(b):
        pltpu.sync_copy(stage_den,
                        acc.at[pl.ds(sid * ROWS_PER_TILE + b * BLK, BLK)])

    pltpu.sync_copy(stage_den.at[pl.ds(0, DEN_PER_TILE)],
                    acc_den.at[pl.ds(sid * DEN_PER_TILE, DEN_PER_TILE)])

    plsc.subcore_barrier()

    wbase = wid * EW

    def load_small(b, p):
        base = wbase + b * BLK
        c1 = pltpu.async_copy(i_hbm.at[pl.ds(base, BLK)], i_b[p], seml)
        c2 = pltpu.async_copy(j_hbm.at[pl.ds(base, BLK)], j_b[p], seml)
        c3 = pltpu.async_copy(bias_hbm.at[pl.ds(base * HEADS, BLK * HEADS)],
                              bias_b[p], seml)
        return c1, c2, c3

    def compute_jdiv(p):
        @pl.loop(0, SUB)
        def _(sb):
            sl = pl.ds(sb * 16, 16)
            jdiv_b[p][sl] = lax.shift_right_logical(j_b[p][sl], 4)

    def den_cells(p, val_from_ex):
        @plsc.parallel_loop(0, BLK // 2, unroll=2)
        def _(pp):
            rowv = 2 * pp + hi8
            jp = plsc.load_gather(j_b[p], [rowv])
            colv = lax.shift_left(jnp.bitwise_and(jp, 15), 3) + lane7
            if val_from_ex:
                plsc.store_scatter(stage_den, [rowv, colv],
                                   bias_b[p][pl.ds(pp * 16, 16)])
            else:
                plsc.store_scatter(stage_den, [rowv, colv], z16)

    def block(b, p, first, last):
        q = 1 - p
        cq = pltpu.async_copy(q_hbm.at[j_b[p]], q_v, semq)
        if not first:
            pltpu.make_async_copy(k_v, acc.at[j_b[q]], semvs).wait()
        ck = pltpu.async_copy(k_hbm.at[i_b[p]], k_v, semk)
        if not first:
            pltpu.make_async_copy(stage_den, acc_den.at[jdiv_b[q]],
                                  semds).wait()
            den_cells(q, False)
        small = None if last else load_small(b + 1, q)
        compute_jdiv(p)
        cq.wait()
        ck.wait()

        @plsc.parallel_loop(0, BLK, unroll=2)
        def _(e):
            for h in range(HEADS):
                prod = (q_v[e, pl.ds(h * DK, 16)] * k_v[e, pl.ds(h * DK, 16)])
                cums = plsc.cumsum(prod)
                plsc.store_scatter(score_v,
                                   [jnp.full((16,), e * 8 + h, jnp.int32)],
                                   cums, mask=lane15)

        cv = pltpu.async_copy(v_hbm.at[i_b[p]], q_v, semv)

        @plsc.parallel_loop(0, BLK * HEADS // 16, unroll=2)
        def _(c):
            sl = pl.ds(c * 16, 16)
            bias_b[p][sl] = jnp.exp(score_v[sl] + bias_b[p][sl])

        den_cells(p, True)
        pltpu.async_copy(stage_den, acc_den.at[jdiv_b[p]], semds, add=True)

        cv.wait()

        @plsc.parallel_loop(0, BLK // 2, unroll=2)
        def _(pp):
            exv = bias_b[p][pl.ds(pp * 16, 16)]
            for sub_e in range(2):
                e = 2 * pp + sub_e
                for h in range(HEADS):
                    bc = exv.at[jnp.full((16,), sub_e * 8 + h, jnp.int32)].get(
                        mode="promise_in_bounds")
                    k_v[e, pl.ds(h * DK, 16)] = bc * q_v[e, pl.ds(h * DK, 16)]

        pltpu.async_copy(k_v, acc.at[j_b[p]], semvs, add=True)
        if small is not None:
            for c in small:
                c.wait()

    for c in load_small(0, 0):
        c.wait()
    block(0, 0, True, False)

    @pl.loop(0, (NBLK - 3) // 2)
    def _(t):
        block(2 * t + 1, 1, False, False)
        block(2 * t + 2, 0, False, False)

    block(NBLK - 2, 1, False, False)
    block(NBLK - 1, 0, False, True)

    pltpu.make_async_copy(k_v, acc.at[j_b[0]], semvs).wait()
    pltpu.make_async_copy(stage_den, acc_den.at[jdiv_b[0]], semds).wait()

    plsc.subcore_barrier()

    pltpu.sync_copy(acc.at[pl.ds(sid * ROWS_PER_TILE, ROWS_PER_TILE)],
                    out_hbm.at[cid, pl.ds(sid * ROWS_PER_TILE, ROWS_PER_TILE)])
    pltpu.sync_copy(acc_den.at[pl.ds(sid * DEN_PER_TILE, DEN_PER_TILE)],
                    outden_hbm.at[cid, pl.ds(sid * DEN_PER_TILE, DEN_PER_TILE)])


_sc_params = pltpu.CompilerParams()
if "needs_layout_passes" in pltpu.CompilerParams.__dataclass_fields__:
    _sc_params = dataclasses.replace(_sc_params, needs_layout_passes=False)

_sc_edge_pass = functools.partial(
    pl.kernel,
    compiler_params=_sc_params,
    out_type=(jax.ShapeDtypeStruct((NC, N_PAD, DIM), jnp.float32),
              jax.ShapeDtypeStruct((NC, NDEN, DIM), jnp.float32)),
    mesh=plsc.VectorSubcoreMesh(core_axis_name="c", subcore_axis_name="s"),
    scratch_types=[
        pltpu.VMEM((BLK,), jnp.int32),
        pltpu.VMEM((BLK,), jnp.int32),
        pltpu.VMEM((BLK,), jnp.int32),
        pltpu.VMEM((BLK,), jnp.int32),
        pltpu.VMEM((BLK,), jnp.int32),
        pltpu.VMEM((BLK,), jnp.int32),
        pltpu.VMEM((BLK * HEADS,), jnp.float32),
        pltpu.VMEM((BLK * HEADS,), jnp.float32),
        pltpu.VMEM((BLK * HEADS,), jnp.float32),
        pltpu.VMEM((BLK, DIM), jnp.float32),
        pltpu.VMEM((BLK, DIM), jnp.float32),
        pltpu.VMEM((BLK, DIM), jnp.float32),
        pltpu.VMEM_SHARED((N_PAD, DIM), jnp.float32),
        pltpu.VMEM_SHARED((NDEN, DIM), jnp.float32),
        pltpu.SemaphoreType.DMA,
        pltpu.SemaphoreType.DMA,
        pltpu.SemaphoreType.DMA,
        pltpu.SemaphoreType.DMA,
        pltpu.SemaphoreType.DMA,
        pltpu.SemaphoreType.DMA,
    ],
)(_sc_body)


def _final_body(np_ref, dp_ref, wo_ref, bo_ref, o_ref):
    node = np_ref[0, :N] + np_ref[1, :N]
    den = dp_ref[0] + dp_ref[1]
    k_i = lax.broadcasted_iota(jnp.int32, (DIM, 16 * DIM), 0)
    q_i = lax.broadcasted_iota(jnp.int32, (DIM, 16 * DIM), 1)
    mm = (k_i == 8 * (q_i // DIM) + (q_i % DIM) // DK).astype(jnp.float32)
    den_exp = lax.dot_general(den, mm, (((1,), (0,)), ((), ())),
                              precision=_HIGH,
                              preferred_element_type=jnp.float32)
    den_exp = den_exp.reshape(N_PAD, DIM)[:N]
    norm = node / (den_exp + 1e-16)
    o_ref[...] = _dotT(norm, wo_ref[...]) + bo_ref[...][None, :]


def _final(node_p, den_p, WO, bO):
    return pl.pallas_call(
        _final_body,
        grid=(1,),
        in_specs=[
            pl.BlockSpec((NC, N_PAD, DIM), lambda i: (0, 0, 0)),
            pl.BlockSpec((NC, NDEN, DIM), lambda i: (0, 0, 0)),
            pl.BlockSpec((DIM, DIM), lambda i: (0, 0)),
            pl.BlockSpec((DIM,), lambda i: (0,)),
        ],
        out_specs=pl.BlockSpec((N, DIM), lambda i: (0, 0)),
        out_shape=jax.ShapeDtypeStruct((N, DIM), jnp.float32),
    )(node_p, den_p, WO, bO)


def kernel(x, edge_index, edge_attr, WQ, bQ, WK, bK, WV, bV, WO, bO,
           W1, b1, W2, b2):
    qs, ks, vs = _proj(x, WQ, bQ, WK, bK, WV, bV)
    bias = _edge_bias(edge_attr, W1, b1, W2, b2)
    i_idx = edge_index[0]
    j_idx = edge_index[1]
    node_p, den_p = _sc_edge_pass(qs, ks, vs, bias, i_idx, j_idx)
    return _final(node_p, den_p, WO, bO)
